# Initial kernel scaffold; baseline (speedup 1.0000x reference)
import jax
import jax.numpy as jnp
from jax.experimental import pallas as pl


def kernel(patches, mask, query_h, W_patch, b_patch, Wq, bq, Wk, bk, Wv, bv):
    raise NotImplementedError("write your pallas kernel here")


# speedup vs baseline: 1.5184x; 1.5184x over previous
import math

import jax
import jax.numpy as jnp
from jax.experimental import pallas as pl
from jax.experimental.pallas import tpu as pltpu

N, K, DIN, H = 16, 4096, 1024, 64


def _fused_kernel(x_ref, maskf_ref, qh_ref, Wp_ref, bp_ref, Wq_ref, bq_ref,
                  Wk_ref, bk_ref, Wv_ref, bv_ref, wsi_ref, attn_ref):
    x = x_ref[0]
    z = x @ Wp_ref[...] + bp_ref[...]
    p = z * 0.5 * (1.0 + jax.lax.erf(z * (1.0 / math.sqrt(2.0))))
    q = qh_ref[0] @ Wq_ref[...] + bq_ref[...]
    k = p @ Wk_ref[...] + bk_ref[...]
    v = p @ Wv_ref[...] + bv_ref[...]
    s = jax.lax.dot_general(q, k, (((1,), (1,)), ((), ())))
    s = s * (1.0 / math.sqrt(H))
    s = jnp.where(maskf_ref[0] > 0, s, -jnp.inf)
    m = jnp.max(s, axis=1, keepdims=True)
    e = jnp.exp(s - m)
    l = jnp.sum(e, axis=1, keepdims=True)
    attn = e / l
    attn_ref[0] = attn
    wsi_ref[0] = attn @ v


@jax.jit
def kernel(patches, mask, query_h, W_patch, b_patch, Wq, bq, Wk, bk, Wv, bv):
    maskf = mask.astype(jnp.float32).reshape(N, 1, K)
    full = lambda shape: pl.BlockSpec(shape, lambda n: (0,) * len(shape))
    wsi, attn = pl.pallas_call(
        _fused_kernel,
        grid=(N,),
        in_specs=[
            pl.BlockSpec((1, K, DIN), lambda n: (n, 0, 0)),
            pl.BlockSpec((1, 1, K), lambda n: (n, 0, 0)),
            pl.BlockSpec((1, 1, H), lambda n: (n, 0, 0)),
            full((DIN, H)),
            full((1, H)),
            full((H, H)), full((1, H)),
            full((H, H)), full((1, H)),
            full((H, H)), full((1, H)),
        ],
        out_specs=[
            pl.BlockSpec((1, 1, H), lambda n: (n, 0, 0)),
            pl.BlockSpec((1, 1, K), lambda n: (n, 0, 0)),
        ],
        out_shape=[
            jax.ShapeDtypeStruct((N, 1, H), jnp.float32),
            jax.ShapeDtypeStruct((N, 1, K), jnp.float32),
        ],
        compiler_params=pltpu.CompilerParams(
            dimension_semantics=("arbitrary",),
        ),
    )(patches, maskf, query_h.reshape(N, 1, H), W_patch, b_patch.reshape(1, H),
      Wq, bq.reshape(1, H), Wk, bk.reshape(1, H), Wv, bv.reshape(1, H))
    return (wsi.reshape(N, H), attn.reshape(N, K))
